# baseline (speedup 1.0000x reference)
import functools

import jax
import jax.numpy as jnp
from jax import lax
from jax.experimental import pallas as pl
from jax.experimental.pallas import tpu as pltpu
from jax.experimental.pallas import tpu_sc as plsc

B, N, C, D = 4, 1024, 8, 512
BN = B * N
TN = 128
NTILES = BN // TN

NC, NS = 2, 16
NW = NC * NS
NPW = BN // NW
EPW = NPW * C
CHUNK = 8
ROWS = CHUNK * C
NCHUNK = NPW // CHUNK


def _coeffs(ch):
    chf = ch.astype(jnp.float32)
    mask = jnp.minimum(chf, 1.0)
    num_sib = jnp.sum(mask, axis=1, keepdims=True)
    jidx = lax.broadcasted_iota(jnp.int32, ch.shape, 1).astype(jnp.float32)
    child_idx = jidx * mask
    denom = jnp.where(num_sib == 1.0, 1.0, num_sib - 1.0)
    single = jnp.where(jidx == 0.0, 0.5, 0.0)
    cr = jnp.where(num_sib == 1.0, single, child_idx / denom)
    cl = (1.0 - cr) * mask
    return cr, cl, mask


def _rep16(x):
    return jnp.broadcast_to(x[:, :, None], (x.shape[0], C, 16)).reshape(
        x.shape[0], C * 16)


def _stage1_body(nodes_ref, ce_ref, ch_ref, wt, wr, wl, bias, wt1, bias1,
                 h0_ref, t1_ref, gidx_ref, crm_ref, clm_ref):
    ch = ch_ref[...]
    cr, cl, mask = _coeffs(ch)
    pre_r = cr[:, 0:1] * ce_ref[:, 0, :]
    pre_l = cl[:, 0:1] * ce_ref[:, 0, :]
    for j in range(1, C):
        cej = ce_ref[:, j, :]
        pre_r = pre_r + cr[:, j:j + 1] * cej
        pre_l = pre_l + cl[:, j:j + 1] * cej
    acc = jnp.dot(nodes_ref[...], wt[...], preferred_element_type=jnp.float32)
    acc = acc + jnp.dot(pre_r, wr[...], preferred_element_type=jnp.float32)
    acc = acc + jnp.dot(pre_l, wl[...], preferred_element_type=jnp.float32)
    h0 = jnp.tanh(acc + bias[...])
    h0_ref[...] = h0
    t1_ref[...] = jnp.dot(h0, wt1[...], preferred_element_type=jnp.float32) + bias1[...]
    b = pl.program_id(0) // (N // TN)
    gidx_ref[...] = ch + b * N
    crm_ref[...] = _rep16(cr * mask)
    clm_ref[...] = _rep16(cl)


def _stage3_body(t1_ref, prer_ref, prel_ref, wr, wl, out_ref):
    acc = t1_ref[...] + jnp.dot(prer_ref[...], wr[...], preferred_element_type=jnp.float32)
    acc = acc + jnp.dot(prel_ref[...], wl[...], preferred_element_type=jnp.float32)
    out_ref[...] = jnp.tanh(acc)


def _row_spec():
    return pl.BlockSpec((TN, D), lambda i: (i, 0))


def _full_spec():
    return pl.BlockSpec((D, D), lambda i: (0, 0))


def _bias_spec():
    return pl.BlockSpec((1, D), lambda i: (0, 0))


def _stage2_sc(h0, gidx, crm, clm):
    mesh = plsc.VectorSubcoreMesh(core_axis_name="c", subcore_axis_name="s")

    @functools.partial(
        pl.kernel,
        mesh=mesh,
        out_type=[jax.ShapeDtypeStruct((BN, D), jnp.float32),
                  jax.ShapeDtypeStruct((BN, D), jnp.float32)],
        scratch_types=[
            pltpu.VMEM((EPW,), jnp.int32),
            pltpu.VMEM((NPW, 128), jnp.float32),
            pltpu.VMEM((NPW, 128), jnp.float32),
            pltpu.VMEM((ROWS, D), jnp.float32),
            pltpu.VMEM((ROWS, D), jnp.float32),
            pltpu.VMEM((CHUNK, D), jnp.float32),
            pltpu.VMEM((CHUNK, D), jnp.float32),
            pltpu.VMEM((CHUNK, D), jnp.float32),
            pltpu.VMEM((CHUNK, D), jnp.float32),
            pltpu.SemaphoreType.DMA,
            pltpu.SemaphoreType.DMA,
            pltpu.SemaphoreType.DMA,
            pltpu.SemaphoreType.DMA,
        ],
    )
    def run(h0_hbm, gidx_hbm, crm_hbm, clm_hbm, prer_hbm, prel_hbm,
            idx_v, crm_v, clm_v, rows0, rows1, outr0, outl0, outr1, outl1,
            gsem0, gsem1, ssem0, ssem1):
        cid = lax.axis_index("c")
        sid = lax.axis_index("s")
        wid = sid * NC + cid
        ebase = wid * EPW
        nbase = wid * NPW
        pltpu.sync_copy(gidx_hbm.at[pl.ds(ebase, EPW)], idx_v)
        pltpu.sync_copy(crm_hbm.at[pl.ds(nbase, NPW)], crm_v)
        pltpu.sync_copy(clm_hbm.at[pl.ds(nbase, NPW)], clm_v)

        rows = (rows0, rows1)
        outr = (outr0, outr1)
        outl = (outl0, outl1)
        gsem = (gsem0, gsem1)
        ssem = (ssem0, ssem1)

        def fire(ck, b):
            pltpu.async_copy(
                h0_hbm.at[idx_v.at[pl.ds(ck * ROWS, ROWS)]], rows[b], gsem[b])

        def wait_rows(b):
            pltpu.make_async_copy(
                h0_hbm.at[pl.ds(0, ROWS)], rows[b], gsem[b]).wait()

        def wait_stores(b):
            pltpu.make_async_copy(
                outr[b], prer_hbm.at[pl.ds(0, CHUNK)], ssem[b]).wait()
            pltpu.make_async_copy(
                outl[b], prel_hbm.at[pl.ds(0, CHUNK)], ssem[b]).wait()

        def compute(ck, b):
            rv, orv, olv = rows[b], outr[b], outl[b]

            def node_body(i, carry2):
                nl = ck * CHUNK + i
                cw = [crm_v[nl, pl.ds(j * 16, 16)] for j in range(C)]
                lw = [clm_v[nl, pl.ds(j * 16, 16)] for j in range(C)]

                def d_body(d, carry3):
                    sl = pl.ds(d * 16, 16)
                    r0 = rv[i * C, sl]
                    accr = r0 * cw[0]
                    accl = r0 * lw[0]
                    for j in range(1, C):
                        rj = rv[i * C + j, sl]
                        accr = accr + rj * cw[j]
                        accl = accl + rj * lw[j]
                    orv[i, sl] = accr
                    olv[i, sl] = accl
                    return carry3

                return lax.fori_loop(0, D // 16, d_body, carry2, unroll=2)

            lax.fori_loop(0, CHUNK, node_body, 0)
            row0 = nbase + ck * CHUNK
            pltpu.async_copy(orv, prer_hbm.at[pl.ds(row0, CHUNK)], ssem[b])
            pltpu.async_copy(olv, prel_hbm.at[pl.ds(row0, CHUNK)], ssem[b])

        fire(0, 0)

        def outer(k, carry):
            fire(2 * k + 1, 1)

            @pl.when(k > 0)
            def _():
                wait_stores(0)

            wait_rows(0)
            compute(2 * k, 0)

            @pl.when(k < NCHUNK // 2 - 1)
            def _():
                fire(2 * k + 2, 0)

            @pl.when(k > 0)
            def _():
                wait_stores(1)

            wait_rows(1)
            compute(2 * k + 1, 1)
            return carry

        lax.fori_loop(0, NCHUNK // 2, outer, 0)
        wait_stores(0)
        wait_stores(1)

    return run(h0, gidx, crm, clm)


def kernel(nodes, children, children_embedding,
           w_t_0, w_r_0, w_l_0, b_0, w_t_1, w_r_1, w_l_1, b_1):
    nodes2 = nodes.reshape(BN, D)
    ce2 = children_embedding.reshape(BN, C, D)
    ch2 = children.reshape(BN, C).astype(jnp.int32)

    h0, t1, gidx, crm_b, clm_b = pl.pallas_call(
        _stage1_body,
        grid=(NTILES,),
        in_specs=[
            _row_spec(),
            pl.BlockSpec((TN, C, D), lambda i: (i, 0, 0)),
            pl.BlockSpec((TN, C), lambda i: (i, 0)),
            _full_spec(), _full_spec(), _full_spec(),
            _bias_spec(),
            _full_spec(),
            _bias_spec(),
        ],
        out_specs=[
            _row_spec(),
            _row_spec(),
            pl.BlockSpec((TN, C), lambda i: (i, 0)),
            pl.BlockSpec((TN, C * 16), lambda i: (i, 0)),
            pl.BlockSpec((TN, C * 16), lambda i: (i, 0)),
        ],
        out_shape=[
            jax.ShapeDtypeStruct((BN, D), jnp.float32),
            jax.ShapeDtypeStruct((BN, D), jnp.float32),
            jax.ShapeDtypeStruct((BN, C), jnp.int32),
            jax.ShapeDtypeStruct((BN, C * 16), jnp.float32),
            jax.ShapeDtypeStruct((BN, C * 16), jnp.float32),
        ],
    )(nodes2, ce2, ch2, w_t_0, w_r_0, w_l_0, b_0.reshape(1, D),
      w_t_1, b_1.reshape(1, D))

    pre_r, pre_l = _stage2_sc(h0, gidx.reshape(BN * C), crm_b, clm_b)

    out = pl.pallas_call(
        _stage3_body,
        grid=(NTILES,),
        in_specs=[
            _row_spec(), _row_spec(), _row_spec(),
            _full_spec(), _full_spec(),
        ],
        out_specs=_row_spec(),
        out_shape=jax.ShapeDtypeStruct((BN, D), jnp.float32),
    )(t1, pre_r, pre_l, w_r_1, w_l_1)

    return out.reshape(B, N, D)


# speedup vs baseline: 1.1059x; 1.1059x over previous
import functools

import jax
import jax.numpy as jnp
from jax import lax
from jax.experimental import pallas as pl
from jax.experimental.pallas import tpu as pltpu
from jax.experimental.pallas import tpu_sc as plsc

B, N, C, D = 4, 1024, 8, 512
BN = B * N
TN = 512
NTILES = BN // TN

NC, NS = 2, 16
NW = NC * NS
NPW = BN // NW
EPW = NPW * C
CHUNK = 8
ROWS = CHUNK * C
NCHUNK = NPW // CHUNK


def _coeffs(ch):
    chf = ch.astype(jnp.float32)
    mask = jnp.minimum(chf, 1.0)
    num_sib = jnp.sum(mask, axis=1, keepdims=True)
    jidx = lax.broadcasted_iota(jnp.int32, ch.shape, 1).astype(jnp.float32)
    child_idx = jidx * mask
    denom = jnp.where(num_sib == 1.0, 1.0, num_sib - 1.0)
    single = jnp.where(jidx == 0.0, 0.5, 0.0)
    cr = jnp.where(num_sib == 1.0, single, child_idx / denom)
    cl = (1.0 - cr) * mask
    return cr, cl, mask


def _rep16(x):
    return jnp.broadcast_to(x[:, :, None], (x.shape[0], C, 16)).reshape(
        x.shape[0], C * 16)


def _stage1_body(nodes_ref, ce_ref, ch_ref, wt, wr, wl, bias, wt1, bias1,
                 h0_ref, t1_ref, gidx_ref, crm_ref, clm_ref):
    ch = ch_ref[...]
    cr, cl, mask = _coeffs(ch)
    pre_r = cr[:, 0:1] * ce_ref[:, 0, :]
    pre_l = cl[:, 0:1] * ce_ref[:, 0, :]
    for j in range(1, C):
        cej = ce_ref[:, j, :]
        pre_r = pre_r + cr[:, j:j + 1] * cej
        pre_l = pre_l + cl[:, j:j + 1] * cej
    acc = jnp.dot(nodes_ref[...], wt[...], preferred_element_type=jnp.float32)
    acc = acc + jnp.dot(pre_r, wr[...], preferred_element_type=jnp.float32)
    acc = acc + jnp.dot(pre_l, wl[...], preferred_element_type=jnp.float32)
    h0 = jnp.tanh(acc + bias[...])
    h0_ref[...] = h0
    t1_ref[...] = jnp.dot(h0, wt1[...], preferred_element_type=jnp.float32) + bias1[...]
    b = pl.program_id(0) // (N // TN)
    gidx_ref[...] = ch + b * N
    crm_ref[...] = _rep16(cr * mask)
    clm_ref[...] = _rep16(cl)


def _stage3_body(t1_ref, prer_ref, prel_ref, wr, wl, out_ref):
    acc = t1_ref[...] + jnp.dot(prer_ref[...], wr[...], preferred_element_type=jnp.float32)
    acc = acc + jnp.dot(prel_ref[...], wl[...], preferred_element_type=jnp.float32)
    out_ref[...] = jnp.tanh(acc)


def _row_spec():
    return pl.BlockSpec((TN, D), lambda i: (i, 0))


def _full_spec():
    return pl.BlockSpec((D, D), lambda i: (0, 0))


def _bias_spec():
    return pl.BlockSpec((1, D), lambda i: (0, 0))


def _stage2_sc(h0, gidx, crm, clm):
    mesh = plsc.VectorSubcoreMesh(core_axis_name="c", subcore_axis_name="s")

    @functools.partial(
        pl.kernel,
        mesh=mesh,
        out_type=[jax.ShapeDtypeStruct((BN, D), jnp.float32),
                  jax.ShapeDtypeStruct((BN, D), jnp.float32)],
        scratch_types=[
            pltpu.VMEM((EPW,), jnp.int32),
            pltpu.VMEM((NPW, 128), jnp.float32),
            pltpu.VMEM((NPW, 128), jnp.float32),
            pltpu.VMEM((ROWS, D), jnp.float32),
            pltpu.VMEM((ROWS, D), jnp.float32),
            pltpu.VMEM((CHUNK, D), jnp.float32),
            pltpu.VMEM((CHUNK, D), jnp.float32),
            pltpu.VMEM((CHUNK, D), jnp.float32),
            pltpu.VMEM((CHUNK, D), jnp.float32),
            pltpu.SemaphoreType.DMA,
            pltpu.SemaphoreType.DMA,
            pltpu.SemaphoreType.DMA,
            pltpu.SemaphoreType.DMA,
        ],
    )
    def run(h0_hbm, gidx_hbm, crm_hbm, clm_hbm, prer_hbm, prel_hbm,
            idx_v, crm_v, clm_v, rows0, rows1, outr0, outl0, outr1, outl1,
            gsem0, gsem1, ssem0, ssem1):
        cid = lax.axis_index("c")
        sid = lax.axis_index("s")
        wid = sid * NC + cid
        ebase = wid * EPW
        nbase = wid * NPW
        pltpu.sync_copy(gidx_hbm.at[pl.ds(ebase, EPW)], idx_v)
        pltpu.sync_copy(crm_hbm.at[pl.ds(nbase, NPW)], crm_v)
        pltpu.sync_copy(clm_hbm.at[pl.ds(nbase, NPW)], clm_v)

        rows = (rows0, rows1)
        outr = (outr0, outr1)
        outl = (outl0, outl1)
        gsem = (gsem0, gsem1)
        ssem = (ssem0, ssem1)

        def fire(ck, b):
            pltpu.async_copy(
                h0_hbm.at[idx_v.at[pl.ds(ck * ROWS, ROWS)]], rows[b], gsem[b])

        def wait_rows(b):
            pltpu.make_async_copy(
                h0_hbm.at[pl.ds(0, ROWS)], rows[b], gsem[b]).wait()

        def wait_stores(b):
            pltpu.make_async_copy(
                outr[b], prer_hbm.at[pl.ds(0, CHUNK)], ssem[b]).wait()
            pltpu.make_async_copy(
                outl[b], prel_hbm.at[pl.ds(0, CHUNK)], ssem[b]).wait()

        def compute(ck, b):
            rv, orv, olv = rows[b], outr[b], outl[b]

            def node_body(i, carry2):
                nl = ck * CHUNK + i
                cw = [crm_v[nl, pl.ds(j * 16, 16)] for j in range(C)]
                lw = [clm_v[nl, pl.ds(j * 16, 16)] for j in range(C)]

                def d_body(d, carry3):
                    sl = pl.ds(d * 16, 16)
                    r0 = rv[i * C, sl]
                    accr = r0 * cw[0]
                    accl = r0 * lw[0]
                    for j in range(1, C):
                        rj = rv[i * C + j, sl]
                        accr = accr + rj * cw[j]
                        accl = accl + rj * lw[j]
                    orv[i, sl] = accr
                    olv[i, sl] = accl
                    return carry3

                return lax.fori_loop(0, D // 16, d_body, carry2, unroll=2)

            lax.fori_loop(0, CHUNK, node_body, 0)
            row0 = nbase + ck * CHUNK
            pltpu.async_copy(orv, prer_hbm.at[pl.ds(row0, CHUNK)], ssem[b])
            pltpu.async_copy(olv, prel_hbm.at[pl.ds(row0, CHUNK)], ssem[b])

        fire(0, 0)

        def outer(k, carry):
            fire(2 * k + 1, 1)

            @pl.when(k > 0)
            def _():
                wait_stores(0)

            wait_rows(0)
            compute(2 * k, 0)

            @pl.when(k < NCHUNK // 2 - 1)
            def _():
                fire(2 * k + 2, 0)

            @pl.when(k > 0)
            def _():
                wait_stores(1)

            wait_rows(1)
            compute(2 * k + 1, 1)
            return carry

        lax.fori_loop(0, NCHUNK // 2, outer, 0)
        wait_stores(0)
        wait_stores(1)

    return run(h0, gidx, crm, clm)


def kernel(nodes, children, children_embedding,
           w_t_0, w_r_0, w_l_0, b_0, w_t_1, w_r_1, w_l_1, b_1):
    nodes2 = nodes.reshape(BN, D)
    ce2 = children_embedding.reshape(BN, C, D)
    ch2 = children.reshape(BN, C).astype(jnp.int32)

    h0, t1, gidx, crm_b, clm_b = pl.pallas_call(
        _stage1_body,
        grid=(NTILES,),
        in_specs=[
            _row_spec(),
            pl.BlockSpec((TN, C, D), lambda i: (i, 0, 0)),
            pl.BlockSpec((TN, C), lambda i: (i, 0)),
            _full_spec(), _full_spec(), _full_spec(),
            _bias_spec(),
            _full_spec(),
            _bias_spec(),
        ],
        out_specs=[
            _row_spec(),
            _row_spec(),
            pl.BlockSpec((TN, C), lambda i: (i, 0)),
            pl.BlockSpec((TN, C * 16), lambda i: (i, 0)),
            pl.BlockSpec((TN, C * 16), lambda i: (i, 0)),
        ],
        out_shape=[
            jax.ShapeDtypeStruct((BN, D), jnp.float32),
            jax.ShapeDtypeStruct((BN, D), jnp.float32),
            jax.ShapeDtypeStruct((BN, C), jnp.int32),
            jax.ShapeDtypeStruct((BN, C * 16), jnp.float32),
            jax.ShapeDtypeStruct((BN, C * 16), jnp.float32),
        ],
    )(nodes2, ce2, ch2, w_t_0, w_r_0, w_l_0, b_0.reshape(1, D),
      w_t_1, b_1.reshape(1, D))

    pre_r, pre_l = _stage2_sc(h0, gidx.reshape(BN * C), crm_b, clm_b)

    out = pl.pallas_call(
        _stage3_body,
        grid=(NTILES,),
        in_specs=[
            _row_spec(), _row_spec(), _row_spec(),
            _full_spec(), _full_spec(),
        ],
        out_specs=_row_spec(),
        out_shape=jax.ShapeDtypeStruct((BN, D), jnp.float32),
    )(t1, pre_r, pre_l, w_r_1, w_l_1)

    return out.reshape(B, N, D)


# speedup vs baseline: 1.1642x; 1.0527x over previous
import functools

import jax
import jax.numpy as jnp
from jax import lax
from jax.experimental import pallas as pl
from jax.experimental.pallas import tpu as pltpu
from jax.experimental.pallas import tpu_sc as plsc

B, N, C, D = 4, 1024, 8, 512
BN = B * N
HB = 2
M = HB * N
TN = 512
NTILES = M // TN

NC, NS = 2, 16
NW = NC * NS
NPW = M // NW
EPW = NPW * C
CHUNK = 8
ROWS = CHUNK * C
NCHUNK = NPW // CHUNK


def _coeffs(ch):
    chf = ch.astype(jnp.float32)
    mask = jnp.minimum(chf, 1.0)
    num_sib = jnp.sum(mask, axis=1, keepdims=True)
    jidx = lax.broadcasted_iota(jnp.int32, ch.shape, 1).astype(jnp.float32)
    child_idx = jidx * mask
    denom = jnp.where(num_sib == 1.0, 1.0, num_sib - 1.0)
    single = jnp.where(jidx == 0.0, 0.5, 0.0)
    cr = jnp.where(num_sib == 1.0, single, child_idx / denom)
    cl = (1.0 - cr) * mask
    return cr, cl, mask


def _rep16(x):
    return jnp.broadcast_to(x[:, :, None], (x.shape[0], C, 16)).reshape(
        x.shape[0], C * 16)


def _stage1_body(nodes_ref, ce_ref, ch_ref, wt, wr, wl, bias, wt1, bias1,
                 h0_ref, t1_ref, gidx_ref, crm_ref, clm_ref):
    ch = ch_ref[...]
    cr, cl, mask = _coeffs(ch)
    pre_r = cr[:, 0:1] * ce_ref[:, 0, :]
    pre_l = cl[:, 0:1] * ce_ref[:, 0, :]
    for j in range(1, C):
        cej = ce_ref[:, j, :]
        pre_r = pre_r + cr[:, j:j + 1] * cej
        pre_l = pre_l + cl[:, j:j + 1] * cej
    acc = jnp.dot(nodes_ref[...], wt[...], preferred_element_type=jnp.float32)
    acc = acc + jnp.dot(pre_r, wr[...], preferred_element_type=jnp.float32)
    acc = acc + jnp.dot(pre_l, wl[...], preferred_element_type=jnp.float32)
    h0 = jnp.tanh(acc + bias[...])
    h0_ref[...] = h0
    t1_ref[...] = jnp.dot(h0, wt1[...], preferred_element_type=jnp.float32) + bias1[...]
    b = pl.program_id(0) // (N // TN)
    gidx_ref[...] = ch + b * N
    crm_ref[...] = _rep16(cr * mask)
    clm_ref[...] = _rep16(cl)


def _stage3_body(t1_ref, prer_ref, prel_ref, wr, wl, out_ref):
    acc = t1_ref[...] + jnp.dot(prer_ref[...], wr[...], preferred_element_type=jnp.float32)
    acc = acc + jnp.dot(prel_ref[...], wl[...], preferred_element_type=jnp.float32)
    out_ref[...] = jnp.tanh(acc)


def _row_spec():
    return pl.BlockSpec((TN, D), lambda i: (i, 0))


def _full_spec():
    return pl.BlockSpec((D, D), lambda i: (0, 0))


def _bias_spec():
    return pl.BlockSpec((1, D), lambda i: (0, 0))


def _stage1(h, nodes2, ce2, ch2, wt0, wr0, wl0, b0r, wt1, b1r):
    off = h * NTILES
    return pl.pallas_call(
        _stage1_body,
        grid=(NTILES,),
        in_specs=[
            pl.BlockSpec((TN, D), lambda i: (i + off, 0)),
            pl.BlockSpec((TN, C, D), lambda i: (i + off, 0, 0)),
            pl.BlockSpec((TN, C), lambda i: (i + off, 0)),
            _full_spec(), _full_spec(), _full_spec(),
            _bias_spec(),
            _full_spec(),
            _bias_spec(),
        ],
        out_specs=[
            _row_spec(),
            _row_spec(),
            pl.BlockSpec((TN, C), lambda i: (i, 0)),
            pl.BlockSpec((TN, C * 16), lambda i: (i, 0)),
            pl.BlockSpec((TN, C * 16), lambda i: (i, 0)),
        ],
        out_shape=[
            jax.ShapeDtypeStruct((M, D), jnp.float32),
            jax.ShapeDtypeStruct((M, D), jnp.float32),
            jax.ShapeDtypeStruct((M, C), jnp.int32),
            jax.ShapeDtypeStruct((M, C * 16), jnp.float32),
            jax.ShapeDtypeStruct((M, C * 16), jnp.float32),
        ],
    )(nodes2, ce2, ch2, wt0, wr0, wl0, b0r, wt1, b1r)


def _stage2_sc(h0, gidx, crm, clm):
    mesh = plsc.VectorSubcoreMesh(core_axis_name="c", subcore_axis_name="s")

    @functools.partial(
        pl.kernel,
        mesh=mesh,
        out_type=[jax.ShapeDtypeStruct((M, D), jnp.float32),
                  jax.ShapeDtypeStruct((M, D), jnp.float32)],
        scratch_types=[
            pltpu.VMEM((EPW,), jnp.int32),
            pltpu.VMEM((NPW, 128), jnp.float32),
            pltpu.VMEM((NPW, 128), jnp.float32),
            pltpu.VMEM((ROWS, D), jnp.float32),
            pltpu.VMEM((ROWS, D), jnp.float32),
            pltpu.VMEM((CHUNK, D), jnp.float32),
            pltpu.VMEM((CHUNK, D), jnp.float32),
            pltpu.VMEM((CHUNK, D), jnp.float32),
            pltpu.VMEM((CHUNK, D), jnp.float32),
            pltpu.SemaphoreType.DMA,
            pltpu.SemaphoreType.DMA,
            pltpu.SemaphoreType.DMA,
            pltpu.SemaphoreType.DMA,
        ],
    )
    def run(h0_hbm, gidx_hbm, crm_hbm, clm_hbm, prer_hbm, prel_hbm,
            idx_v, crm_v, clm_v, rows0, rows1, outr0, outl0, outr1, outl1,
            gsem0, gsem1, ssem0, ssem1):
        cid = lax.axis_index("c")
        sid = lax.axis_index("s")
        wid = sid * NC + cid
        ebase = wid * EPW
        nbase = wid * NPW
        pltpu.sync_copy(gidx_hbm.at[pl.ds(ebase, EPW)], idx_v)
        pltpu.sync_copy(crm_hbm.at[pl.ds(nbase, NPW)], crm_v)
        pltpu.sync_copy(clm_hbm.at[pl.ds(nbase, NPW)], clm_v)

        rows = (rows0, rows1)
        outr = (outr0, outr1)
        outl = (outl0, outl1)
        gsem = (gsem0, gsem1)
        ssem = (ssem0, ssem1)

        def fire(ck, b):
            pltpu.async_copy(
                h0_hbm.at[idx_v.at[pl.ds(ck * ROWS, ROWS)]], rows[b], gsem[b])

        def wait_rows(b):
            pltpu.make_async_copy(
                h0_hbm.at[pl.ds(0, ROWS)], rows[b], gsem[b]).wait()

        def wait_stores(b):
            pltpu.make_async_copy(
                outr[b], prer_hbm.at[pl.ds(0, CHUNK)], ssem[b]).wait()
            pltpu.make_async_copy(
                outl[b], prel_hbm.at[pl.ds(0, CHUNK)], ssem[b]).wait()

        def compute(ck, b):
            rv, orv, olv = rows[b], outr[b], outl[b]

            def node_body(i, carry2):
                nl = ck * CHUNK + i
                cw = [crm_v[nl, pl.ds(j * 16, 16)] for j in range(C)]
                lw = [clm_v[nl, pl.ds(j * 16, 16)] for j in range(C)]

                def d_body(d, carry3):
                    sl = pl.ds(d * 16, 16)
                    r0 = rv[i * C, sl]
                    accr = r0 * cw[0]
                    accl = r0 * lw[0]
                    for j in range(1, C):
                        rj = rv[i * C + j, sl]
                        accr = accr + rj * cw[j]
                        accl = accl + rj * lw[j]
                    orv[i, sl] = accr
                    olv[i, sl] = accl
                    return carry3

                return lax.fori_loop(0, D // 16, d_body, carry2, unroll=2)

            lax.fori_loop(0, CHUNK, node_body, 0)
            row0 = nbase + ck * CHUNK
            pltpu.async_copy(orv, prer_hbm.at[pl.ds(row0, CHUNK)], ssem[b])
            pltpu.async_copy(olv, prel_hbm.at[pl.ds(row0, CHUNK)], ssem[b])

        fire(0, 0)

        def outer(k, carry):
            fire(2 * k + 1, 1)

            @pl.when(k > 0)
            def _():
                wait_stores(0)

            wait_rows(0)
            compute(2 * k, 0)

            @pl.when(k < NCHUNK // 2 - 1)
            def _():
                fire(2 * k + 2, 0)

            @pl.when(k > 0)
            def _():
                wait_stores(1)

            wait_rows(1)
            compute(2 * k + 1, 1)
            return carry

        lax.fori_loop(0, NCHUNK // 2, outer, 0)
        wait_stores(0)
        wait_stores(1)

    return run(h0, gidx, crm, clm)


def _stage3(t1, pre_r, pre_l, wr1, wl1):
    return pl.pallas_call(
        _stage3_body,
        grid=(NTILES,),
        in_specs=[
            _row_spec(), _row_spec(), _row_spec(),
            _full_spec(), _full_spec(),
        ],
        out_specs=_row_spec(),
        out_shape=jax.ShapeDtypeStruct((M, D), jnp.float32),
    )(t1, pre_r, pre_l, wr1, wl1)


def kernel(nodes, children, children_embedding,
           w_t_0, w_r_0, w_l_0, b_0, w_t_1, w_r_1, w_l_1, b_1):
    nodes2 = nodes.reshape(BN, D)
    ce2 = children_embedding.reshape(BN, C, D)
    ch2 = children.reshape(BN, C).astype(jnp.int32)
    b0r = b_0.reshape(1, D)
    b1r = b_1.reshape(1, D)

    stage1_res = []
    for h in range(2):
        stage1_res.append(_stage1(
            h, nodes2, ce2, ch2,
            w_t_0, w_r_0, w_l_0, b0r, w_t_1, b1r))

    outs = []
    for h in range(2):
        h0, t1, gidx, crm_b, clm_b = stage1_res[h]
        pre_r, pre_l = _stage2_sc(h0, gidx.reshape(M * C), crm_b, clm_b)
        outs.append(_stage3(t1, pre_r, pre_l, w_r_1, w_l_1))

    return jnp.concatenate(outs, axis=0).reshape(B, N, D)


# speedup vs baseline: 1.2206x; 1.0484x over previous
import functools

import jax
import jax.numpy as jnp
from jax import lax
from jax.experimental import pallas as pl
from jax.experimental.pallas import tpu as pltpu
from jax.experimental.pallas import tpu_sc as plsc

B, N, C, D = 4, 1024, 8, 512
BN = B * N
HB = 2
M = HB * N
TN = 512
NTILES = M // TN

NC, NS = 2, 16
NW = NC * NS
NPW = M // NW
EPW = NPW * C
CHUNK = 8
ROWS = CHUNK * C
NCHUNK = NPW // CHUNK


def _coeffs(ch):
    chf = ch.astype(jnp.float32)
    mask = jnp.minimum(chf, 1.0)
    num_sib = jnp.sum(mask, axis=1, keepdims=True)
    jidx = lax.broadcasted_iota(jnp.int32, ch.shape, 1).astype(jnp.float32)
    child_idx = jidx * mask
    denom = jnp.where(num_sib == 1.0, 1.0, num_sib - 1.0)
    single = jnp.where(jidx == 0.0, 0.5, 0.0)
    cr = jnp.where(num_sib == 1.0, single, child_idx / denom)
    cl = (1.0 - cr) * mask
    return cr, cl, mask


def _rep16(x):
    return jnp.broadcast_to(x[:, :, None], (x.shape[0], C, 16)).reshape(
        x.shape[0], C * 16)


def _stage1_body(nodes_ref, ce_ref, ch_ref, wt, wr, wl, bias, wt1, bias1,
                 h0_ref, t1_ref, gidx_ref, crm_ref, clm_ref):
    ch = ch_ref[...]
    cr, cl, mask = _coeffs(ch)
    pre_r = cr[:, 0:1] * ce_ref[:, 0, :]
    pre_l = cl[:, 0:1] * ce_ref[:, 0, :]
    for j in range(1, C):
        cej = ce_ref[:, j, :]
        pre_r = pre_r + cr[:, j:j + 1] * cej
        pre_l = pre_l + cl[:, j:j + 1] * cej
    acc = jnp.dot(nodes_ref[...], wt[...], preferred_element_type=jnp.float32)
    acc = acc + jnp.dot(pre_r, wr[...], preferred_element_type=jnp.float32)
    acc = acc + jnp.dot(pre_l, wl[...], preferred_element_type=jnp.float32)
    h0 = jnp.tanh(acc + bias[...])
    h0_ref[...] = h0
    t1_ref[...] = jnp.dot(h0, wt1[...], preferred_element_type=jnp.float32) + bias1[...]
    b = pl.program_id(0) // (N // TN)
    gidx_ref[...] = ch + b * N
    crm_ref[...] = _rep16(cr * mask)
    clm_ref[...] = _rep16(cl)


def _stage3_body(t1_ref, prer_ref, prel_ref, wr, wl, out_ref):
    acc = t1_ref[...] + jnp.dot(prer_ref[...], wr[...], preferred_element_type=jnp.float32)
    acc = acc + jnp.dot(prel_ref[...], wl[...], preferred_element_type=jnp.float32)
    out_ref[...] = jnp.tanh(acc)


def _row_spec():
    return pl.BlockSpec((TN, D), lambda i: (i, 0))


def _full_spec():
    return pl.BlockSpec((D, D), lambda i: (0, 0))


def _bias_spec():
    return pl.BlockSpec((1, D), lambda i: (0, 0))


def _stage1(h, nodes2, ce2, ch2, wt0, wr0, wl0, b0r, wt1, b1r):
    off = h * NTILES
    return pl.pallas_call(
        _stage1_body,
        grid=(NTILES,),
        in_specs=[
            pl.BlockSpec((TN, D), lambda i: (i + off, 0)),
            pl.BlockSpec((TN, C, D), lambda i: (i + off, 0, 0)),
            pl.BlockSpec((TN, C), lambda i: (i + off, 0)),
            _full_spec(), _full_spec(), _full_spec(),
            _bias_spec(),
            _full_spec(),
            _bias_spec(),
        ],
        out_specs=[
            _row_spec(),
            _row_spec(),
            pl.BlockSpec((TN, C), lambda i: (i, 0)),
            pl.BlockSpec((TN, C * 16), lambda i: (i, 0)),
            pl.BlockSpec((TN, C * 16), lambda i: (i, 0)),
        ],
        out_shape=[
            jax.ShapeDtypeStruct((M, D), jnp.float32),
            jax.ShapeDtypeStruct((M, D), jnp.float32),
            jax.ShapeDtypeStruct((M, C), jnp.int32),
            jax.ShapeDtypeStruct((M, C * 16), jnp.float32),
            jax.ShapeDtypeStruct((M, C * 16), jnp.float32),
        ],
    )(nodes2, ce2, ch2, wt0, wr0, wl0, b0r, wt1, b1r)


def _stage2_sc(h0, gidx, crm, clm):
    mesh = plsc.VectorSubcoreMesh(core_axis_name="c", subcore_axis_name="s")

    @functools.partial(
        pl.kernel,
        mesh=mesh,
        out_type=[jax.ShapeDtypeStruct((M, D), jnp.float32),
                  jax.ShapeDtypeStruct((M, D), jnp.float32)],
        scratch_types=[
            pltpu.VMEM((EPW,), jnp.int32),
            pltpu.VMEM((NPW, 128), jnp.float32),
            pltpu.VMEM((NPW, 128), jnp.float32),
            pltpu.VMEM((ROWS, D), jnp.float32),
            pltpu.VMEM((ROWS, D), jnp.float32),
            pltpu.VMEM((CHUNK, D), jnp.float32),
            pltpu.VMEM((CHUNK, D), jnp.float32),
            pltpu.VMEM((CHUNK, D), jnp.float32),
            pltpu.VMEM((CHUNK, D), jnp.float32),
            pltpu.SemaphoreType.DMA,
            pltpu.SemaphoreType.DMA,
            pltpu.SemaphoreType.DMA,
            pltpu.SemaphoreType.DMA,
        ],
    )
    def run(h0_hbm, gidx_hbm, crm_hbm, clm_hbm, prer_hbm, prel_hbm,
            idx_v, crm_v, clm_v, rows0, rows1, outr0, outl0, outr1, outl1,
            gsem0, gsem1, ssem0, ssem1):
        cid = lax.axis_index("c")
        sid = lax.axis_index("s")
        wid = sid * NC + cid
        ebase = wid * EPW
        nbase = wid * NPW
        pltpu.sync_copy(gidx_hbm.at[pl.ds(ebase, EPW)], idx_v)
        pltpu.sync_copy(crm_hbm.at[pl.ds(nbase, NPW)], crm_v)
        pltpu.sync_copy(clm_hbm.at[pl.ds(nbase, NPW)], clm_v)

        rows = (rows0, rows1)
        outr = (outr0, outr1)
        outl = (outl0, outl1)
        gsem = (gsem0, gsem1)
        ssem = (ssem0, ssem1)

        def fire(ck, b):
            pltpu.async_copy(
                h0_hbm.at[idx_v.at[pl.ds(ck * ROWS, ROWS)]], rows[b], gsem[b])

        def wait_rows(b):
            pltpu.make_async_copy(
                h0_hbm.at[pl.ds(0, ROWS)], rows[b], gsem[b]).wait()

        def wait_stores(b):
            pltpu.make_async_copy(
                outr[b], prer_hbm.at[pl.ds(0, CHUNK)], ssem[b]).wait()
            pltpu.make_async_copy(
                outl[b], prel_hbm.at[pl.ds(0, CHUNK)], ssem[b]).wait()

        def compute(ck, b):
            rv, orv, olv = rows[b], outr[b], outl[b]

            def node_body(i, carry2):
                nl = ck * CHUNK + i
                cw = [crm_v[nl, pl.ds(j * 16, 16)] for j in range(C)]
                lw = [clm_v[nl, pl.ds(j * 16, 16)] for j in range(C)]

                def d_body(d, carry3):
                    sl = pl.ds(d * 16, 16)
                    r0 = rv[i * C, sl]
                    accr = r0 * cw[0]
                    accl = r0 * lw[0]
                    for j in range(1, C):
                        rj = rv[i * C + j, sl]
                        accr = accr + rj * cw[j]
                        accl = accl + rj * lw[j]
                    orv[i, sl] = accr
                    olv[i, sl] = accl
                    return carry3

                return lax.fori_loop(0, D // 16, d_body, carry2, unroll=2)

            lax.fori_loop(0, CHUNK, node_body, 0)
            row0 = nbase + ck * CHUNK
            pltpu.async_copy(orv, prer_hbm.at[pl.ds(row0, CHUNK)], ssem[b])
            pltpu.async_copy(olv, prel_hbm.at[pl.ds(row0, CHUNK)], ssem[b])

        fire(0, 0)

        def outer(k, carry):
            fire(2 * k + 1, 1)

            @pl.when(k > 0)
            def _():
                wait_stores(0)

            wait_rows(0)
            compute(2 * k, 0)

            @pl.when(k < NCHUNK // 2 - 1)
            def _():
                fire(2 * k + 2, 0)

            @pl.when(k > 0)
            def _():
                wait_stores(1)

            wait_rows(1)
            compute(2 * k + 1, 1)
            return carry

        lax.fori_loop(0, NCHUNK // 2, outer, 0)
        wait_stores(0)
        wait_stores(1)

    return run(h0, gidx, crm, clm)


def _stage3(h, t1, pre_r, pre_l, wr1, wl1, prev):
    off = h * NTILES
    body = _stage3_body if h == 0 else (
        lambda p, a, b_, c_, d_, e_, o: _stage3_body(a, b_, c_, d_, e_, o))
    in_specs = [
        _row_spec(), _row_spec(), _row_spec(),
        _full_spec(), _full_spec(),
    ]
    args = (t1, pre_r, pre_l, wr1, wl1)
    kwargs = {}
    if h == 1:
        in_specs = [pl.BlockSpec(memory_space=pl.ANY)] + in_specs
        args = (prev,) + args
        kwargs = dict(input_output_aliases={0: 0})
    return pl.pallas_call(
        body,
        grid=(NTILES,),
        in_specs=in_specs,
        out_specs=pl.BlockSpec((TN, D), lambda i: (i + off, 0)),
        out_shape=jax.ShapeDtypeStruct((BN, D), jnp.float32),
        **kwargs,
    )(*args)


def kernel(nodes, children, children_embedding,
           w_t_0, w_r_0, w_l_0, b_0, w_t_1, w_r_1, w_l_1, b_1):
    nodes2 = nodes.reshape(BN, D)
    ce2 = children_embedding.reshape(BN, C, D)
    ch2 = children.reshape(BN, C).astype(jnp.int32)
    b0r = b_0.reshape(1, D)
    b1r = b_1.reshape(1, D)

    stage1_res = []
    for h in range(2):
        stage1_res.append(_stage1(
            h, nodes2, ce2, ch2,
            w_t_0, w_r_0, w_l_0, b0r, w_t_1, b1r))

    out = None
    for h in range(2):
        h0, t1, gidx, crm_b, clm_b = stage1_res[h]
        pre_r, pre_l = _stage2_sc(h0, gidx.reshape(M * C), crm_b, clm_b)
        out = _stage3(h, t1, pre_r, pre_l, w_r_1, w_l_1, out)

    return out.reshape(B, N, D)
